# Initial kernel scaffold; baseline (speedup 1.0000x reference)
import jax
import jax.numpy as jnp
from jax.experimental import pallas as pl


def kernel(x, tiles):
    raise NotImplementedError("write your pallas kernel here")


# speedup vs baseline: 2.7896x; 2.7896x over previous
import jax
import jax.numpy as jnp
from jax import lax
from jax.experimental import pallas as pl
from jax.experimental.pallas import tpu as pltpu
from jax.experimental.pallas import tpu_sc as plsc

DOUT = 64
N_BAGS = 16384
BAG = 50
NW = 32
BAGS_PER_W = N_BAGS // NW
GRP = 8
N_GRP = BAGS_PER_W // GRP
IDX_PER_GRP = GRP * BAG


def _emb_body(x_hbm, w_hbm, out_hbm, idx_v, rows_v, out_v, sem):
    c = lax.axis_index("c")
    s = lax.axis_index("s")
    wid = s * 2 + c

    def group_body(g, carry):
        bag0 = wid * BAGS_PER_W + g * GRP
        pltpu.sync_copy(x_hbm.at[pl.ds(bag0 * BAG, IDX_PER_GRP)], idx_v)
        pltpu.async_copy(w_hbm.at[idx_v], rows_v, sem).wait()
        for b in range(GRP):
            def row_body(j, acc):
                r = b * BAG + j
                return tuple(acc[k] + rows_v[r, pl.ds(k * 16, 16)]
                             for k in range(4))
            acc = lax.fori_loop(
                0, BAG, row_body,
                tuple(jnp.zeros((16,), jnp.float32) for _ in range(4)))
            for k in range(4):
                out_v[b, pl.ds(k * 16, 16)] = acc[k]
        pltpu.sync_copy(out_v, out_hbm.at[pl.ds(bag0, GRP)])
        return carry

    lax.fori_loop(0, N_GRP, group_body, 0)


def kernel(x, tiles):
    dout = tiles.shape[-1]
    zeroed = tiles.at[0, 0, 3, :].set(0.0)
    zeroed = zeroed.at[:, :, :, 0, 0, 3, :].set(0.0)
    w = jnp.concatenate(
        [zeroed.reshape(768 * 768, dout),
         jnp.zeros((1, dout), tiles.dtype)], axis=0)
    x_flat = x.reshape(-1)
    f = pl.kernel(
        _emb_body,
        out_type=jax.ShapeDtypeStruct((N_BAGS, DOUT), jnp.float32),
        mesh=plsc.VectorSubcoreMesh(core_axis_name="c", subcore_axis_name="s"),
        scratch_types=[
            pltpu.VMEM((IDX_PER_GRP,), jnp.int32),
            pltpu.VMEM((IDX_PER_GRP, DOUT), jnp.float32),
            pltpu.VMEM((GRP, DOUT), jnp.float32),
            pltpu.SemaphoreType.DMA,
        ],
        compiler_params=pltpu.CompilerParams(use_tc_tiling_on_sc=False),
    )
    return f(x_flat, w)


# speedup vs baseline: 5.0426x; 1.8077x over previous
import jax
import jax.numpy as jnp
from jax import lax
from jax.experimental import pallas as pl
from jax.experimental.pallas import tpu as pltpu
from jax.experimental.pallas import tpu_sc as plsc

DOUT = 64
N_BAGS = 16384
BAG = 50
NW = 32
BAGS_PER_W = N_BAGS // NW
GRP = 8
N_GRP = BAGS_PER_W // GRP
IDX_PER_GRP = GRP * BAG
N_VEC = IDX_PER_GRP // 16

ZERO_ROW = 768 * 768
A3_LO, A3_HI = 3 * 768, 4 * 768


def _emb_body(x_hbm, w_hbm, out_hbm, idx_raw, idx_g, rows_v, out_v,
              row0_v, sem):
    c = lax.axis_index("c")
    s = lax.axis_index("s")
    wid = s * 2 + c
    pltpu.sync_copy(w_hbm.at[pl.ds(0, 8)], row0_v)
    lane = lax.iota(jnp.int32, 16)

    def group_body(g, carry):
        bag0 = wid * BAGS_PER_W + g * GRP
        pltpu.sync_copy(x_hbm.at[pl.ds(bag0 * BAG, IDX_PER_GRP)], idx_raw)
        cnts = [jnp.zeros((16,), jnp.int32)] * GRP
        for v in range(N_VEC):
            t = idx_raw[pl.ds(v * 16, 16)]
            idx2 = jnp.where(t >= ZERO_ROW, 3, t)
            lo = idx2 & 255
            hi = lax.shift_right_logical(idx2, 8)
            q = lax.shift_right_logical(hi * 21846, 16)
            r3 = hi - q * 3
            inv = ((lo == 3) & (r3 == 0)) | ((idx2 >= A3_LO) & (idx2 < A3_HI))
            idx_g[pl.ds(v * 16, 16)] = jnp.where(inv, 0, idx2)
            start = v * 16
            for b in range(start // 50, (start + 15) // 50 + 1):
                s0 = max(start, b * 50) - start
                e0 = min(start + 16, (b + 1) * 50) - start
                if s0 == 0 and e0 == 16:
                    seg = inv
                else:
                    seg = inv & (lane >= s0) & (lane < e0)
                cnts[b] = cnts[b] + jnp.where(seg, jnp.int32(1), jnp.int32(0))
        pltpu.async_copy(w_hbm.at[idx_g], rows_v, sem).wait()
        for b in range(GRP):
            def row_body(j, acc):
                r = b * BAG + 2 * j
                acc = tuple(acc[k] + rows_v[r, pl.ds(k * 16, 16)]
                            for k in range(4))
                return tuple(acc[k] + rows_v[r + 1, pl.ds(k * 16, 16)]
                             for k in range(4))
            acc = lax.fori_loop(
                0, BAG // 2, row_body,
                tuple(jnp.zeros((16,), jnp.float32) for _ in range(4)))
            tot = cnts[b]
            for sh in (8, 4, 2, 1):
                perm = lane ^ sh
                g = lax.gather(
                    tot, perm.reshape(16, 1),
                    lax.GatherDimensionNumbers(
                        offset_dims=(), collapsed_slice_dims=(0,),
                        start_index_map=(0,)),
                    (1,), mode=lax.GatherScatterMode.PROMISE_IN_BOUNDS)
                tot = tot + g
            cf = tot.astype(jnp.float32)
            for k in range(4):
                out_v[b, pl.ds(k * 16, 16)] = (
                    acc[k] - cf * row0_v[0, pl.ds(k * 16, 16)])
        pltpu.sync_copy(out_v, out_hbm.at[pl.ds(bag0, GRP)])
        return carry

    lax.fori_loop(0, N_GRP, group_body, 0)


def kernel(x, tiles):
    w = tiles.reshape(768 * 768, DOUT)
    x_flat = x.reshape(-1)
    f = pl.kernel(
        _emb_body,
        out_type=jax.ShapeDtypeStruct((N_BAGS, DOUT), jnp.float32),
        mesh=plsc.VectorSubcoreMesh(core_axis_name="c", subcore_axis_name="s"),
        scratch_types=[
            pltpu.VMEM((IDX_PER_GRP,), jnp.int32),
            pltpu.VMEM((IDX_PER_GRP,), jnp.int32),
            pltpu.VMEM((IDX_PER_GRP, DOUT), jnp.float32),
            pltpu.VMEM((GRP, DOUT), jnp.float32),
            pltpu.VMEM((8, DOUT), jnp.float32),
            pltpu.SemaphoreType.DMA,
        ],
        compiler_params=pltpu.CompilerParams(use_tc_tiling_on_sc=False),
    )
    return f(x_flat, w)


# speedup vs baseline: 6.2540x; 1.2402x over previous
import jax
import jax.numpy as jnp
from jax import lax
from jax.experimental import pallas as pl
from jax.experimental.pallas import tpu as pltpu
from jax.experimental.pallas import tpu_sc as plsc

DOUT = 64
N_BAGS = 16384
BAG = 50
NW = 32
BAGS_PER_W = N_BAGS // NW
GRP = 8
N_GRP = BAGS_PER_W // GRP
IDX_PER_GRP = GRP * BAG
N_VEC = IDX_PER_GRP // 16

ZERO_ROW = 768 * 768
A3_LO, A3_HI = 3 * 768, 4 * 768


def _emb_body(x_hbm, w_hbm, out_hbm,
              xb0, xb1, ig0, ig1, rows0, rows1, ob0, ob1, cf0, cf1,
              row0_v, sg0, sg1, so0, so1):
    c = lax.axis_index("c")
    s = lax.axis_index("s")
    wid = s * 2 + c
    bag_base = wid * BAGS_PER_W
    pltpu.sync_copy(w_hbm.at[pl.ds(0, 8)], row0_v)
    lane = lax.iota(jnp.int32, 16)

    def transform(g, xb, ig, cf):
        pltpu.sync_copy(x_hbm.at[pl.ds((bag_base + g * GRP) * BAG,
                                       IDX_PER_GRP)], xb)
        cnts = [jnp.zeros((16,), jnp.int32)] * GRP
        for v in range(N_VEC):
            t = xb[pl.ds(v * 16, 16)]
            idx2 = jnp.where(t >= ZERO_ROW, 3, t)
            lo = idx2 & 255
            hi = lax.shift_right_logical(idx2, 8)
            q = lax.shift_right_logical(hi * 21846, 16)
            r3 = hi - q * 3
            inv = ((lo == 3) & (r3 == 0)) | ((idx2 >= A3_LO) & (idx2 < A3_HI))
            ig[pl.ds(v * 16, 16)] = jnp.where(inv, 0, idx2)
            start = v * 16
            for b in range(start // 50, (start + 15) // 50 + 1):
                s0 = max(start, b * 50) - start
                e0 = min(start + 16, (b + 1) * 50) - start
                if s0 == 0 and e0 == 16:
                    seg = inv
                else:
                    seg = inv & (lane >= s0) & (lane < e0)
                cnts[b] = cnts[b] + jnp.where(seg, jnp.int32(1), jnp.int32(0))
        dnums = lax.GatherDimensionNumbers(
            offset_dims=(), collapsed_slice_dims=(0,), start_index_map=(0,))
        for b in range(GRP):
            tot = cnts[b]
            for sh in (8, 4, 2, 1):
                g_ = lax.gather(tot, (lane ^ sh).reshape(16, 1), dnums, (1,),
                                mode=lax.GatherScatterMode.PROMISE_IN_BOUNDS)
                tot = tot + g_
            cf[pl.ds(b * 16, 16)] = tot.astype(jnp.float32)

    def gather_start(ig, rows, sem):
        pltpu.make_async_copy(w_hbm.at[ig], rows, sem).start()

    def gather_wait(ig, rows, sem):
        pltpu.make_async_copy(w_hbm.at[ig], rows, sem).wait()

    def accumulate(rows, cf, ob):
        for b in range(GRP):
            def row_body(j, acc):
                r = b * BAG + 2 * j
                acc = tuple(acc[k] + rows[r, pl.ds(k * 16, 16)]
                            for k in range(4))
                return tuple(acc[k] + rows[r + 1, pl.ds(k * 16, 16)]
                             for k in range(4))
            acc = lax.fori_loop(
                0, BAG // 2, row_body,
                tuple(jnp.zeros((16,), jnp.float32) for _ in range(4)))
            cfb = cf[pl.ds(b * 16, 16)]
            for k in range(4):
                ob[b, pl.ds(k * 16, 16)] = (
                    acc[k] - cfb * row0_v[0, pl.ds(k * 16, 16)])

    def out_copy(g, ob, sem):
        return pltpu.make_async_copy(
            ob, out_hbm.at[pl.ds(bag_base + g * GRP, GRP)], sem)

    transform(0, xb0, ig0, cf0)
    gather_start(ig0, rows0, sg0)

    def pair_body(i, carry):
        ga = 2 * i
        transform(ga + 1, xb1, ig1, cf1)
        gather_start(ig1, rows1, sg1)
        gather_wait(ig0, rows0, sg0)

        @pl.when(i > 0)
        def _():
            out_copy(ga - 2, ob0, so0).wait()
        accumulate(rows0, cf0, ob0)
        out_copy(ga, ob0, so0).start()

        @pl.when(i < N_GRP // 2 - 1)
        def _():
            transform(ga + 2, xb0, ig0, cf0)
            gather_start(ig0, rows0, sg0)
        gather_wait(ig1, rows1, sg1)

        @pl.when(i > 0)
        def _():
            out_copy(ga - 1, ob1, so1).wait()
        accumulate(rows1, cf1, ob1)
        out_copy(ga + 1, ob1, so1).start()
        return carry

    lax.fori_loop(0, N_GRP // 2, pair_body, 0)
    out_copy(N_GRP - 2, ob0, so0).wait()
    out_copy(N_GRP - 1, ob1, so1).wait()


def kernel(x, tiles):
    w = tiles.reshape(768 * 768, DOUT)
    x_flat = x.reshape(-1)
    f = pl.kernel(
        _emb_body,
        out_type=jax.ShapeDtypeStruct((N_BAGS, DOUT), jnp.float32),
        mesh=plsc.VectorSubcoreMesh(core_axis_name="c", subcore_axis_name="s"),
        scratch_types=[
            pltpu.VMEM((IDX_PER_GRP,), jnp.int32),
            pltpu.VMEM((IDX_PER_GRP,), jnp.int32),
            pltpu.VMEM((IDX_PER_GRP,), jnp.int32),
            pltpu.VMEM((IDX_PER_GRP,), jnp.int32),
            pltpu.VMEM((IDX_PER_GRP, DOUT), jnp.float32),
            pltpu.VMEM((IDX_PER_GRP, DOUT), jnp.float32),
            pltpu.VMEM((GRP, DOUT), jnp.float32),
            pltpu.VMEM((GRP, DOUT), jnp.float32),
            pltpu.VMEM((GRP * 16,), jnp.float32),
            pltpu.VMEM((GRP * 16,), jnp.float32),
            pltpu.VMEM((8, DOUT), jnp.float32),
            pltpu.SemaphoreType.DMA,
            pltpu.SemaphoreType.DMA,
            pltpu.SemaphoreType.DMA,
            pltpu.SemaphoreType.DMA,
        ],
        compiler_params=pltpu.CompilerParams(use_tc_tiling_on_sc=False),
    )
    return f(x_flat, w)


# speedup vs baseline: 6.2686x; 1.0023x over previous
import jax
import jax.numpy as jnp
from jax import lax
from jax.experimental import pallas as pl
from jax.experimental.pallas import tpu as pltpu
from jax.experimental.pallas import tpu_sc as plsc

DOUT = 64
N_BAGS = 16384
BAG = 50
NW = 32
BAGS_PER_W = N_BAGS // NW
GRP = 8
N_GRP = BAGS_PER_W // GRP
IDX_PER_GRP = GRP * BAG
N_VEC = IDX_PER_GRP // 16

ZERO_ROW = 768 * 768
A3_LO, A3_HI = 3 * 768, 4 * 768


def _emb_body(x_hbm, w_hbm, out_hbm,
              xb0, xb1, ig0, ig1, rows0, rows1, ob0, ob1, cf0, cf1,
              row0_v, sg0, sg1, so0, so1, sx0, sx1):
    c = lax.axis_index("c")
    s = lax.axis_index("s")
    wid = s * 2 + c
    bag_base = wid * BAGS_PER_W
    pltpu.sync_copy(w_hbm.at[pl.ds(0, 8)], row0_v)
    lane = lax.iota(jnp.int32, 16)

    def idx_copy(g, xb, sem):
        return pltpu.make_async_copy(
            x_hbm.at[pl.ds((bag_base + g * GRP) * BAG, IDX_PER_GRP)], xb, sem)

    def classify(g, xb, ig, cf):
        cnts = [jnp.zeros((16,), jnp.int32)] * GRP
        for v in range(N_VEC):
            t = xb[pl.ds(v * 16, 16)]
            idx2 = jnp.where(t >= ZERO_ROW, 3, t)
            lo = idx2 & 255
            hi = lax.shift_right_logical(idx2, 8)
            q = lax.shift_right_logical(hi * 21846, 16)
            r3 = hi - q * 3
            inv = ((lo == 3) & (r3 == 0)) | ((idx2 >= A3_LO) & (idx2 < A3_HI))
            ig[pl.ds(v * 16, 16)] = jnp.where(inv, 0, idx2)
            start = v * 16
            for b in range(start // 50, (start + 15) // 50 + 1):
                s0 = max(start, b * 50) - start
                e0 = min(start + 16, (b + 1) * 50) - start
                if s0 == 0 and e0 == 16:
                    seg = inv
                else:
                    seg = inv & (lane >= s0) & (lane < e0)
                cnts[b] = cnts[b] + jnp.where(seg, jnp.int32(1), jnp.int32(0))
        dnums = lax.GatherDimensionNumbers(
            offset_dims=(), collapsed_slice_dims=(0,), start_index_map=(0,))
        for b in range(GRP):
            tot = cnts[b]
            for sh in (8, 4, 2, 1):
                g_ = lax.gather(tot, (lane ^ sh).reshape(16, 1), dnums, (1,),
                                mode=lax.GatherScatterMode.PROMISE_IN_BOUNDS)
                tot = tot + g_
            cf[pl.ds(b * 16, 16)] = tot.astype(jnp.float32)

    def gather_start(ig, rows, sem):
        pltpu.make_async_copy(w_hbm.at[ig], rows, sem).start()

    def gather_wait(ig, rows, sem):
        pltpu.make_async_copy(w_hbm.at[ig], rows, sem).wait()

    def accumulate(rows, cf, ob):
        for b in range(GRP):
            def row_body(j, acc):
                r = b * BAG + 5 * j
                for u in range(5):
                    acc = tuple(acc[k] + rows[r + u, pl.ds(k * 16, 16)]
                                for k in range(4))
                return acc
            acc = lax.fori_loop(
                0, BAG // 5, row_body,
                tuple(jnp.zeros((16,), jnp.float32) for _ in range(4)))
            cfb = cf[pl.ds(b * 16, 16)]
            for k in range(4):
                ob[b, pl.ds(k * 16, 16)] = (
                    acc[k] - cfb * row0_v[0, pl.ds(k * 16, 16)])

    def out_copy(g, ob, sem):
        return pltpu.make_async_copy(
            ob, out_hbm.at[pl.ds(bag_base + g * GRP, GRP)], sem)

    idx_copy(0, xb0, sx0).start()
    idx_copy(1, xb1, sx1).start()
    idx_copy(0, xb0, sx0).wait()
    classify(0, xb0, ig0, cf0)
    gather_start(ig0, rows0, sg0)
    idx_copy(2, xb0, sx0).start()

    def pair_body(i, carry):
        ga = 2 * i
        idx_copy(ga + 1, xb1, sx1).wait()
        classify(ga + 1, xb1, ig1, cf1)
        gather_start(ig1, rows1, sg1)

        @pl.when(i < N_GRP // 2 - 1)
        def _():
            idx_copy(ga + 3, xb1, sx1).start()
        gather_wait(ig0, rows0, sg0)

        @pl.when(i > 0)
        def _():
            out_copy(ga - 2, ob0, so0).wait()
        accumulate(rows0, cf0, ob0)
        out_copy(ga, ob0, so0).start()

        @pl.when(i < N_GRP // 2 - 1)
        def _():
            idx_copy(ga + 2, xb0, sx0).wait()
            classify(ga + 2, xb0, ig0, cf0)
            gather_start(ig0, rows0, sg0)

        @pl.when(i < N_GRP // 2 - 2)
        def _():
            idx_copy(ga + 4, xb0, sx0).start()
        gather_wait(ig1, rows1, sg1)

        @pl.when(i > 0)
        def _():
            out_copy(ga - 1, ob1, so1).wait()
        accumulate(rows1, cf1, ob1)
        out_copy(ga + 1, ob1, so1).start()
        return carry

    lax.fori_loop(0, N_GRP // 2, pair_body, 0)
    out_copy(N_GRP - 2, ob0, so0).wait()
    out_copy(N_GRP - 1, ob1, so1).wait()


def kernel(x, tiles):
    w = tiles.reshape(768 * 768, DOUT)
    x_flat = x.reshape(-1)
    f = pl.kernel(
        _emb_body,
        out_type=jax.ShapeDtypeStruct((N_BAGS, DOUT), jnp.float32),
        mesh=plsc.VectorSubcoreMesh(core_axis_name="c", subcore_axis_name="s"),
        scratch_types=[
            pltpu.VMEM((IDX_PER_GRP,), jnp.int32),
            pltpu.VMEM((IDX_PER_GRP,), jnp.int32),
            pltpu.VMEM((IDX_PER_GRP,), jnp.int32),
            pltpu.VMEM((IDX_PER_GRP,), jnp.int32),
            pltpu.VMEM((IDX_PER_GRP, DOUT), jnp.float32),
            pltpu.VMEM((IDX_PER_GRP, DOUT), jnp.float32),
            pltpu.VMEM((GRP, DOUT), jnp.float32),
            pltpu.VMEM((GRP, DOUT), jnp.float32),
            pltpu.VMEM((GRP * 16,), jnp.float32),
            pltpu.VMEM((GRP * 16,), jnp.float32),
            pltpu.VMEM((8, DOUT), jnp.float32),
            pltpu.SemaphoreType.DMA,
            pltpu.SemaphoreType.DMA,
            pltpu.SemaphoreType.DMA,
            pltpu.SemaphoreType.DMA,
            pltpu.SemaphoreType.DMA,
            pltpu.SemaphoreType.DMA,
        ],
        compiler_params=pltpu.CompilerParams(use_tc_tiling_on_sc=False),
    )
    return f(x_flat, w)
